# baseline (speedup 1.0000x reference)
import functools

import jax
import jax.numpy as jnp
from jax import lax
from jax.experimental import pallas as pl
from jax.experimental.pallas import tpu as pltpu
from jax.experimental.pallas import tpu_sc as plsc

N = 10000
E = 640000
NG = 64
HID = 128
OUT = 512

NP = 10240
NC, NS = 2, 16
NW = NC * NS
K = 128
NCHUNK = 160
EPT = NCHUNK * K
EP = NW * EPT
RPT = NP // NS
NB = 4
RB = 2048


def _mesh():
    return plsc.VectorSubcoreMesh(core_axis_name="c", subcore_axis_name="s")


def _build_deg():
    GRP = 8

    @functools.partial(
        pl.kernel,
        out_type=jax.ShapeDtypeStruct((NC * NP, 16), jnp.float32),
        mesh=_mesh(),
        compiler_params=pltpu.CompilerParams(use_tc_tiling_on_sc=False),
        scratch_types=[
            pltpu.VMEM((NCHUNK, K), jnp.int32),
            pltpu.VMEM((K, 16), jnp.float32),
            pltpu.VMEM_SHARED((NP, 16), jnp.float32),
            pltpu.SemaphoreType.DMA,
        ],
    )
    def deg_kernel(dst3, ones_hbm, out_hbm, didx, ones_v, acc, sem):
        cid = lax.axis_index("c")
        sid = lax.axis_index("s")
        r0 = sid * RPT
        for t in range(RPT // K):
            pltpu.sync_copy(ones_hbm, acc.at[pl.ds(r0 + t * K, K)])
        pltpu.sync_copy(ones_hbm, ones_v)
        pltpu.sync_copy(dst3.at[cid * NS + sid], didx)
        plsc.subcore_barrier()

        def group(g, c):
            base = g * GRP
            ds = [pltpu.async_copy(ones_v, acc.at[didx.at[base + b]], sem,
                                   add=True)
                  for b in range(GRP)]
            for d in ds:
                d.wait()
            return c

        lax.fori_loop(0, NCHUNK // GRP, group, 0)
        plsc.subcore_barrier()
        pltpu.sync_copy(acc.at[pl.ds(r0, RPT)],
                        out_hbm.at[pl.ds(cid * NP + r0, RPT)])

    return deg_kernel


def _build_agg(F, k, slab, nb):
    nchunk = EPT // k
    nslab = nchunk // slab

    @functools.partial(
        pl.kernel,
        out_type=jax.ShapeDtypeStruct((NC * NP, F), jnp.float32),
        mesh=_mesh(),
        compiler_params=pltpu.CompilerParams(use_tc_tiling_on_sc=False),
        scratch_types=[
            pltpu.VMEM((slab, k), jnp.int32),
            pltpu.VMEM((slab, k), jnp.int32),
            [pltpu.VMEM((k, F), jnp.float32)] * nb,
            pltpu.VMEM_SHARED((NP, F), jnp.float32),
            pltpu.SemaphoreType.DMA,
            pltpu.SemaphoreType.DMA,
        ],
    )
    def agg_kernel(src4, dst4, p_hbm, out_hbm,
                   sidx, didx, rows, acc, gsem, ssem):
        cid = lax.axis_index("c")
        sid = lax.axis_index("s")
        wid = cid * NS + sid
        r0 = sid * RPT
        pltpu.sync_copy(p_hbm.at[pl.ds(r0, RPT)], acc.at[pl.ds(r0, RPT)])
        plsc.subcore_barrier()

        def group(g, c):
            base = g * nb
            gds = [pltpu.async_copy(p_hbm.at[sidx.at[base + b]], rows[b],
                                    gsem)
                   for b in range(nb)]
            for d in gds:
                d.wait()
            sds = [pltpu.async_copy(rows[b], acc.at[didx.at[base + b]], ssem,
                                    add=True)
                   for b in range(nb)]
            for d in sds:
                d.wait()
            return c

        def slab_body(s, c):
            pltpu.sync_copy(src4.at[wid, s], sidx)
            pltpu.sync_copy(dst4.at[wid, s], didx)
            lax.fori_loop(0, slab // nb, group, 0)
            return c

        lax.fori_loop(0, nslab, slab_body, 0)
        plsc.subcore_barrier()
        pltpu.sync_copy(acc.at[pl.ds(r0, RPT)],
                        out_hbm.at[pl.ds(cid * NP + r0, RPT)])

    return agg_kernel


def _build_agg_pipe(F, k, slab, nbk):
    nchunk = EPT // k
    nslab = nchunk // slab
    npair = slab // (2 * nbk)

    @functools.partial(
        pl.kernel,
        out_type=jax.ShapeDtypeStruct((NC * NP, F), jnp.float32),
        mesh=_mesh(),
        compiler_params=pltpu.CompilerParams(use_tc_tiling_on_sc=False),
        scratch_types=[
            pltpu.VMEM((slab, k), jnp.int32),
            pltpu.VMEM((slab, k), jnp.int32),
            [pltpu.VMEM((k, F), jnp.float32)] * (2 * nbk),
            pltpu.VMEM((k, F), jnp.float32),
            pltpu.VMEM_SHARED((NP, F), jnp.float32),
            pltpu.SemaphoreType.DMA,
            pltpu.SemaphoreType.DMA,
            pltpu.SemaphoreType.DMA,
        ],
    )
    def agg_kernel(src4, dst4, p_hbm, out_hbm,
                   sidx, didx, rows, dummy, acc, gsem, ssa, ssb):
        cid = lax.axis_index("c")
        sid = lax.axis_index("s")
        wid = cid * NS + sid
        r0 = sid * RPT
        pltpu.sync_copy(p_hbm.at[pl.ds(r0, RPT)], acc.at[pl.ds(r0, RPT)])
        plsc.subcore_barrier()

        def bank(base, off, sem):
            for i in range(nbk):
                pltpu.make_async_copy(p_hbm.at[pl.ds(0, k)],
                                      rows[off + i], sem).wait()
            gds = [pltpu.async_copy(p_hbm.at[sidx.at[base + i]],
                                    rows[off + i], gsem)
                   for i in range(nbk)]
            for d in gds:
                d.wait()
            for i in range(nbk):
                pltpu.async_copy(rows[off + i], acc.at[didx.at[base + i]],
                                 sem, add=True)

        def pair(j, c):
            base = j * 2 * nbk
            bank(base, 0, ssa)
            bank(base + nbk, nbk, ssb)
            return c

        def slab_body(s, c):
            pltpu.sync_copy(src4.at[wid, s], sidx)
            pltpu.sync_copy(dst4.at[wid, s], didx)
            for i in range(nbk):
                pltpu.async_copy(p_hbm.at[pl.ds(0, k)], dummy, ssa)
                pltpu.async_copy(p_hbm.at[pl.ds(0, k)], dummy, ssb)
            lax.fori_loop(0, npair, pair, 0)
            for i in range(nbk):
                pltpu.make_async_copy(p_hbm.at[pl.ds(0, k)],
                                      rows[i], ssa).wait()
                pltpu.make_async_copy(p_hbm.at[pl.ds(0, k)],
                                      rows[nbk + i], ssb).wait()
            return c

        lax.fori_loop(0, nslab, slab_body, 0)
        plsc.subcore_barrier()
        pltpu.sync_copy(acc.at[pl.ds(r0, RPT)],
                        out_hbm.at[pl.ds(cid * NP + r0, RPT)])

    return agg_kernel


def _build_norm():
    def body(dd, xr, dinv_o, p1_o):
        deg = dd[:NP, :] + dd[NP:, :] - 1.0
        dinv = lax.rsqrt(deg)
        dinv_o[...] = dinv
        p1_o[...] = xr[...] * dinv

    return pl.pallas_call(
        body,
        out_shape=(jax.ShapeDtypeStruct((NP, 16), jnp.float32),
                   jax.ShapeDtypeStruct((NP, 16), jnp.float32)),
    )


def _ab_specs(F):
    nb_off = NP // RB
    return [pl.BlockSpec((RB, F), lambda i: (i, 0)),
            pl.BlockSpec((RB, F), lambda i, _o=nb_off: (i + _o, 0))]


def _build_mm(F_in, F_out):
    def body(aa, ab, p, dv, W, b, o):
        d = dv[...][:, :1]
        m = (aa[...] + ab[...] - p[...]) * d
        h = jnp.dot(m, W[...], preferred_element_type=jnp.float32) + b[...]
        o[...] = jnp.maximum(h, 0.0) * d

    rb = pl.BlockSpec((RB, F_in), lambda i: (i, 0))
    return pl.pallas_call(
        body,
        grid=(NP // RB,),
        in_specs=_ab_specs(F_in) + [rb,
                  pl.BlockSpec((RB, 16), lambda i: (i, 0)),
                  pl.BlockSpec((F_in, F_out), lambda i: (0, 0)),
                  pl.BlockSpec((1, F_out), lambda i: (0, 0))],
        out_specs=pl.BlockSpec((RB, F_out), lambda i: (i, 0)),
        out_shape=jax.ShapeDtypeStruct((NP, F_out), jnp.float32),
    )


def _build_final():
    G = NP // RB

    def body(aa, ab, p, dv, bt, W, b, o, seg, cnt):
        i = pl.program_id(0)

        @pl.when(i == 0)
        def _():
            seg[...] = jnp.zeros_like(seg)
            cnt[...] = jnp.zeros_like(cnt)

        g = (aa[...] + ab[...] - p[...]) * dv[...][:, :1]
        rows = i * RB + lax.broadcasted_iota(jnp.int32, (RB, 1), 0)
        valid = (rows < N).astype(jnp.float32)
        oh = (bt[...] == lax.broadcasted_iota(jnp.int32, (1, NG), 1))
        oh = oh.astype(jnp.float32) * valid
        seg[...] += lax.dot_general(oh, g, (((0,), (0,)), ((), ())),
                                    preferred_element_type=jnp.float32)
        cnt[...] += lax.dot_general(oh, valid, (((0,), (0,)), ((), ())),
                                    preferred_element_type=jnp.float32)

        @pl.when(i == G - 1)
        def _():
            pooled = seg[...] / jnp.maximum(cnt[...], 1.0)
            o[...] = jnp.dot(pooled, W[...],
                             preferred_element_type=jnp.float32) + b[...]

    rb128 = pl.BlockSpec((RB, HID), lambda i: (i, 0))
    return pl.pallas_call(
        body,
        grid=(G,),
        in_specs=_ab_specs(HID) + [rb128,
                  pl.BlockSpec((RB, 16), lambda i: (i, 0)),
                  pl.BlockSpec((RB, 1), lambda i: (i, 0)),
                  pl.BlockSpec((HID, OUT), lambda i: (0, 0)),
                  pl.BlockSpec((1, OUT), lambda i: (0, 0))],
        out_specs=pl.BlockSpec((NG, OUT), lambda i: (0, 0)),
        out_shape=jax.ShapeDtypeStruct((NG, OUT), jnp.float32),
        scratch_shapes=[pltpu.VMEM((NG, HID), jnp.float32),
                        pltpu.VMEM((NG, 1), jnp.float32)],
    )


_deg_k = _build_deg()
_agg16 = _build_agg(16, k=K, slab=NCHUNK, nb=8)
_agg128 = _build_agg_pipe(HID, k=64, slab=40, nbk=2)
_norm_k = _build_norm()
_mm1 = _build_mm(16, HID)
_mm2 = _build_mm(HID, HID)
_final_k = _build_final()


def kernel(x, edge_index, batch, W1, b1, W2, b2, W3, b3):
    f32 = jnp.float32
    pad_r = jnp.arange(EP - E, dtype=jnp.int32)
    srcf = jnp.concatenate([edge_index[0], pad_r % N])
    dstf = jnp.concatenate([edge_index[1], N + pad_r % (NP - N)])
    dst3 = dstf.reshape(NW, NCHUNK, K)
    src4a = srcf.reshape(NW, 1, NCHUNK, K)
    dst4a = dstf.reshape(NW, 1, NCHUNK, K)
    src4b = srcf.reshape(NW, 8, 40, 64)
    dst4b = dstf.reshape(NW, 8, 40, 64)
    x16 = jnp.zeros((NP, 16), f32).at[:N, :8].set(x)
    batchp = jnp.pad(batch, (0, NP - N)).reshape(NP, 1)
    ones16 = jnp.ones((K, 16), f32)
    W1p = jnp.zeros((16, HID), f32).at[:8].set(W1)

    deg = _deg_k(dst3, ones16)
    dinv, p1 = _norm_k(deg, x16)
    a1 = _agg16(src4a, dst4a, p1)
    p2 = _mm1(a1, a1, p1, dinv, W1p, b1.reshape(1, HID))
    a2 = _agg128(src4b, dst4b, p2)
    p3 = _mm2(a2, a2, p2, dinv, W2, b2.reshape(1, HID))
    a3 = _agg128(src4b, dst4b, p3)
    return _final_k(a3, a3, p3, dinv, batchp, W3, b3.reshape(1, OUT))


# speedup vs baseline: 1.0516x; 1.0516x over previous
import functools

import jax
import jax.numpy as jnp
from jax import lax
from jax.experimental import pallas as pl
from jax.experimental.pallas import tpu as pltpu
from jax.experimental.pallas import tpu_sc as plsc

N = 10000
E = 640000
NG = 64
HID = 128
OUT = 512

NP = 10240
NC, NS = 2, 16
NW = NC * NS
K = 128
NCHUNK = 160
EPT = NCHUNK * K
EP = NW * EPT
RPT = NP // NS
NB = 4
RB = 2048


def _mesh():
    return plsc.VectorSubcoreMesh(core_axis_name="c", subcore_axis_name="s")


def _build_deg():
    GRP = 8

    @functools.partial(
        pl.kernel,
        out_type=jax.ShapeDtypeStruct((NC * NP, 16), jnp.float32),
        mesh=_mesh(),
        compiler_params=pltpu.CompilerParams(use_tc_tiling_on_sc=False),
        scratch_types=[
            pltpu.VMEM((NCHUNK, K), jnp.int32),
            pltpu.VMEM((K, 16), jnp.float32),
            pltpu.VMEM_SHARED((NP, 16), jnp.float32),
            pltpu.SemaphoreType.DMA,
        ],
    )
    def deg_kernel(dst3, ones_hbm, out_hbm, didx, ones_v, acc, sem):
        cid = lax.axis_index("c")
        sid = lax.axis_index("s")
        r0 = sid * RPT
        for t in range(RPT // K):
            pltpu.sync_copy(ones_hbm, acc.at[pl.ds(r0 + t * K, K)])
        pltpu.sync_copy(ones_hbm, ones_v)
        pltpu.sync_copy(dst3.at[cid * NS + sid], didx)
        plsc.subcore_barrier()

        def group(g, c):
            base = g * GRP
            ds = [pltpu.async_copy(ones_v, acc.at[didx.at[base + b]], sem,
                                   add=True)
                  for b in range(GRP)]
            for d in ds:
                d.wait()
            return c

        lax.fori_loop(0, NCHUNK // GRP, group, 0)
        plsc.subcore_barrier()
        pltpu.sync_copy(acc.at[pl.ds(r0, RPT)],
                        out_hbm.at[pl.ds(cid * NP + r0, RPT)])

    return deg_kernel


def _build_agg(F, k, slab, nb):
    nchunk = EPT // k
    nslab = nchunk // slab

    @functools.partial(
        pl.kernel,
        out_type=jax.ShapeDtypeStruct((NC * NP, F), jnp.float32),
        mesh=_mesh(),
        compiler_params=pltpu.CompilerParams(use_tc_tiling_on_sc=False),
        scratch_types=[
            pltpu.VMEM((slab, k), jnp.int32),
            pltpu.VMEM((slab, k), jnp.int32),
            [pltpu.VMEM((k, F), jnp.float32)] * nb,
            pltpu.VMEM_SHARED((NP, F), jnp.float32),
            pltpu.SemaphoreType.DMA,
            pltpu.SemaphoreType.DMA,
        ],
    )
    def agg_kernel(src4, dst4, p_hbm, out_hbm,
                   sidx, didx, rows, acc, gsem, ssem):
        cid = lax.axis_index("c")
        sid = lax.axis_index("s")
        wid = cid * NS + sid
        r0 = sid * RPT
        pltpu.sync_copy(p_hbm.at[pl.ds(r0, RPT)], acc.at[pl.ds(r0, RPT)])
        plsc.subcore_barrier()

        def group(g, c):
            base = g * nb
            gds = [pltpu.async_copy(p_hbm.at[sidx.at[base + b]], rows[b],
                                    gsem)
                   for b in range(nb)]
            for d in gds:
                d.wait()
            sds = [pltpu.async_copy(rows[b], acc.at[didx.at[base + b]], ssem,
                                    add=True)
                   for b in range(nb)]
            for d in sds:
                d.wait()
            return c

        def slab_body(s, c):
            pltpu.sync_copy(src4.at[wid, s], sidx)
            pltpu.sync_copy(dst4.at[wid, s], didx)
            lax.fori_loop(0, slab // nb, group, 0)
            return c

        lax.fori_loop(0, nslab, slab_body, 0)
        plsc.subcore_barrier()
        pltpu.sync_copy(acc.at[pl.ds(r0, RPT)],
                        out_hbm.at[pl.ds(cid * NP + r0, RPT)])

    return agg_kernel


def _build_norm():
    def body(dd, xr, dinv_o, p1_o):
        deg = dd[:NP, :] + dd[NP:, :] - 1.0
        dinv = lax.rsqrt(deg)
        dinv_o[...] = dinv
        p1_o[...] = xr[...] * dinv

    return pl.pallas_call(
        body,
        out_shape=(jax.ShapeDtypeStruct((NP, 16), jnp.float32),
                   jax.ShapeDtypeStruct((NP, 16), jnp.float32)),
    )


def _ab_specs(F):
    nb_off = NP // RB
    return [pl.BlockSpec((RB, F), lambda i: (i, 0)),
            pl.BlockSpec((RB, F), lambda i, _o=nb_off: (i + _o, 0))]


def _build_mm(F_in, F_out):
    def body(aa, ab, p, dv, W, b, o):
        d = dv[...][:, :1]
        m = (aa[...] + ab[...] - p[...]) * d
        h = jnp.dot(m, W[...], preferred_element_type=jnp.float32) + b[...]
        o[...] = jnp.maximum(h, 0.0) * d

    rb = pl.BlockSpec((RB, F_in), lambda i: (i, 0))
    return pl.pallas_call(
        body,
        grid=(NP // RB,),
        in_specs=_ab_specs(F_in) + [rb,
                  pl.BlockSpec((RB, 16), lambda i: (i, 0)),
                  pl.BlockSpec((F_in, F_out), lambda i: (0, 0)),
                  pl.BlockSpec((1, F_out), lambda i: (0, 0))],
        out_specs=pl.BlockSpec((RB, F_out), lambda i: (i, 0)),
        out_shape=jax.ShapeDtypeStruct((NP, F_out), jnp.float32),
    )


def _build_final():
    G = NP // RB

    def body(aa, ab, p, dv, bt, W, b, o, seg, cnt):
        i = pl.program_id(0)

        @pl.when(i == 0)
        def _():
            seg[...] = jnp.zeros_like(seg)
            cnt[...] = jnp.zeros_like(cnt)

        g = (aa[...] + ab[...] - p[...]) * dv[...][:, :1]
        rows = i * RB + lax.broadcasted_iota(jnp.int32, (RB, 1), 0)
        valid = (rows < N).astype(jnp.float32)
        oh = (bt[...] == lax.broadcasted_iota(jnp.int32, (1, NG), 1))
        oh = oh.astype(jnp.float32) * valid
        seg[...] += lax.dot_general(oh, g, (((0,), (0,)), ((), ())),
                                    preferred_element_type=jnp.float32)
        cnt[...] += lax.dot_general(oh, valid, (((0,), (0,)), ((), ())),
                                    preferred_element_type=jnp.float32)

        @pl.when(i == G - 1)
        def _():
            pooled = seg[...] / jnp.maximum(cnt[...], 1.0)
            o[...] = jnp.dot(pooled, W[...],
                             preferred_element_type=jnp.float32) + b[...]

    rb128 = pl.BlockSpec((RB, HID), lambda i: (i, 0))
    return pl.pallas_call(
        body,
        grid=(G,),
        in_specs=_ab_specs(HID) + [rb128,
                  pl.BlockSpec((RB, 16), lambda i: (i, 0)),
                  pl.BlockSpec((RB, 1), lambda i: (i, 0)),
                  pl.BlockSpec((HID, OUT), lambda i: (0, 0)),
                  pl.BlockSpec((1, OUT), lambda i: (0, 0))],
        out_specs=pl.BlockSpec((NG, OUT), lambda i: (0, 0)),
        out_shape=jax.ShapeDtypeStruct((NG, OUT), jnp.float32),
        scratch_shapes=[pltpu.VMEM((NG, HID), jnp.float32),
                        pltpu.VMEM((NG, 1), jnp.float32)],
    )


_deg_k = _build_deg()
_agg16 = _build_agg(16, k=K, slab=NCHUNK, nb=8)
_agg128 = _build_agg(HID, k=64, slab=40, nb=5)
_norm_k = _build_norm()
_mm1 = _build_mm(16, HID)
_mm2 = _build_mm(HID, HID)
_final_k = _build_final()


def kernel(x, edge_index, batch, W1, b1, W2, b2, W3, b3):
    f32 = jnp.float32
    pad_r = jnp.arange(EP - E, dtype=jnp.int32)
    srcf = jnp.concatenate([edge_index[0], pad_r % N])
    dstf = jnp.concatenate([edge_index[1], N + pad_r % (NP - N)])
    dst3 = dstf.reshape(NW, NCHUNK, K)
    src4a = srcf.reshape(NW, 1, NCHUNK, K)
    dst4a = dstf.reshape(NW, 1, NCHUNK, K)
    src4b = srcf.reshape(NW, 8, 40, 64)
    dst4b = dstf.reshape(NW, 8, 40, 64)
    x16 = jnp.zeros((NP, 16), f32).at[:N, :8].set(x)
    batchp = jnp.pad(batch, (0, NP - N)).reshape(NP, 1)
    ones16 = jnp.ones((K, 16), f32)
    W1p = jnp.zeros((16, HID), f32).at[:8].set(W1)

    deg = _deg_k(dst3, ones16)
    dinv, p1 = _norm_k(deg, x16)
    a1 = _agg16(src4a, dst4a, p1)
    p2 = _mm1(a1, a1, p1, dinv, W1p, b1.reshape(1, HID))
    a2 = _agg128(src4b, dst4b, p2)
    p3 = _mm2(a2, a2, p2, dinv, W2, b2.reshape(1, HID))
    a3 = _agg128(src4b, dst4b, p3)
    return _final_k(a3, a3, p3, dinv, batchp, W3, b3.reshape(1, OUT))
